# baseline (speedup 1.0000x reference)
import math

import jax
import jax.numpy as jnp
import numpy as np
from jax import lax
from jax.experimental import pallas as pl
from jax.experimental.pallas import tpu as pltpu
from jax.experimental.pallas import tpu_sc as plsc

NUM_LEVELS = 16
MIN_RES = 16
MAX_RES = 2048
EMB_DIM = 4
HASHMAP = 524288
B = 1048576
F = NUM_LEVELS * EMB_DIM

_RES = np.round(
    np.logspace(math.log10(float(MIN_RES)), math.log10(float(MAX_RES)), NUM_LEVELS)
).astype(np.int32)
_RES8 = [((int(r) + 7) // 8) * 8 for r in _RES]
_OFFS = np.concatenate([[0], np.cumsum(_RES8)[:-1]]).astype(np.int32)
R_TOTAL = int(sum(_RES8))
R_PAD = 7424

NC, NS = 2, 16
NW = NC * NS
PT = B // NW
CHUNK = 512
NGRP = CHUNK // 16
PBLK = B // 128


def _tab_word(lvl, r, d):
    return lvl * (HASHMAP * EMB_DIM) + (r // 128) * 512 + d * 128 + (r % 128)


def _compact_word_indices() -> np.ndarray:
    cidx = np.zeros((2 * EMB_DIM * R_PAD,), dtype=np.int64)
    for shift in range(2):
        for lvl in range(NUM_LEVELS):
            r = int(_RES[lvl])
            i = np.arange(r, dtype=np.int64)
            isrc = np.minimum(i + shift, r - 1)
            h = ((isrc * 73856093) ^ (lvl * 19349663)) & (HASHMAP - 1)
            for d in range(EMB_DIM):
                o = shift * EMB_DIM * R_PAD + d * R_PAD + int(_OFFS[lvl])
                cidx[o : o + r] = _tab_word(lvl, h, d)
    return cidx.astype(np.int32)


_CIDX = _compact_word_indices()
NCW = EMB_DIM * R_PAD
_GCHUNK = 128
_GBATCH = 16

_GK = [
    [d * R_PAD + int(_OFFS[lvl]) for d in range(EMB_DIM)] for lvl in range(NUM_LEVELS)
]
_OK = [[((lvl * EMB_DIM + d) // 8) * 4096 + ((lvl * EMB_DIM + d) % 8) * 128
        for d in range(EMB_DIM)] for lvl in range(NUM_LEVELS)]

OUT_WORDS = B * F


NCH = PT // CHUNK
_OBLK = 4096


def _body(
    tab_hbm,
    cidx_hbm,
    x_hbm,
    out_hbm,
    cbuf_v,
    compact_v,
    x_v0,
    x_v1,
    out_v0,
    out_v1,
    sem_g,
    sem_x,
    sem_o0,
    sem_o1,
):
    cid = lax.axis_index("c")
    sid = lax.axis_index("s")
    wid = sid * NC + cid

    def gather_step(j, carry):
        o = j * (_GCHUNK * _GBATCH)
        pltpu.sync_copy(cidx_hbm.at[pl.ds(o, _GCHUNK * _GBATCH)], cbuf_v)
        copies = []
        for b in range(_GBATCH):
            copies.append(
                pltpu.async_copy(
                    tab_hbm.at[cbuf_v.at[pl.ds(b * _GCHUNK, _GCHUNK)]],
                    compact_v.at[pl.ds(o + b * _GCHUNK, _GCHUNK)],
                    sem_g,
                )
            )
        for cp in copies:
            cp.wait()
        return carry

    lax.fori_loop(0, 2 * NCW // (_GCHUNK * _GBATCH), gather_step, 0)

    iota = lax.iota(jnp.int32, 16)
    base_pt = wid * PT
    base_blk = wid * (PT // 128)

    x_bufs = (x_v0, x_v1)
    out_bufs = (out_v0, out_v1)
    out_sems = (sem_o0, sem_o1)

    def x_copy(c, buf):
        return pltpu.make_async_copy(
            x_hbm.at[pl.ds(base_pt + c * CHUNK, CHUNK)], buf, sem_x
        )

    def out_copy(c, ch, buf, sem):
        dst = ch * (PBLK * 1024) + (base_blk + c * (CHUNK // 128)) * 1024
        return pltpu.make_async_copy(
            buf.at[pl.ds(ch * _OBLK, _OBLK)], out_hbm.at[pl.ds(dst, _OBLK)], sem
        )

    x_copy(0, x_v0).start()

    def super_body(cc, carry):
        for b in range(2):
            c = cc * 2 + b
            xb = x_bufs[b]
            ob = out_bufs[b]
            osem = out_sems[b]

            x_copy(c, xb).wait()

            @pl.when(c + 1 < NCH)
            def _prefetch():
                x_copy(c + 1, x_bufs[1 - b]).start()

            @pl.when(cc >= 1)
            def _drain_prev():
                for ch in range(F // 8):
                    out_copy(c, ch, ob, osem).wait()

            g0 = [
                [compact_v.at[pl.ds(_GK[lvl][d], NCW - _GK[lvl][d])] for d in range(EMB_DIM)]
                for lvl in range(NUM_LEVELS)
            ]
            g1 = [
                [
                    compact_v.at[pl.ds(NCW + _GK[lvl][d], NCW - _GK[lvl][d])]
                    for d in range(EMB_DIM)
                ]
                for lvl in range(NUM_LEVELS)
            ]
            st = [
                [ob.at[pl.ds(_OK[lvl][d], CHUNK * F - _OK[lvl][d])] for d in range(EMB_DIM)]
                for lvl in range(NUM_LEVELS)
            ]

            @plsc.parallel_loop(0, NGRP, 2, unroll=1)
            def grp_body(g):
                gg = (g, g + 1)
                xcs, sgs = {}, {}
                for k in range(2):
                    xv = xb[pl.ds((g + k) * 16, 16)]
                    xcs[k] = jnp.minimum(
                        jnp.maximum(xv, jnp.float32(0.0)), jnp.float32(1.0)
                    )
                    sgs[k] = ((g + k) >> 3) * 1024 + ((g + k) & 7) * 16
                for lvl in range(NUM_LEVELS):
                    rl = int(_RES[lvl])
                    i0s, ws = {}, {}
                    for k in range(2):
                        t = xcs[k] * jnp.float32(rl - 1)
                        i0s[k] = t.astype(jnp.int32)
                        ws[k] = t - i0s[k].astype(jnp.float32)
                    e0s = {
                        (k, d): plsc.load_gather(g0[lvl][d], [i0s[k]])
                        for k in range(2)
                        for d in range(EMB_DIM)
                    }
                    e1s = {
                        (k, d): plsc.load_gather(g1[lvl][d], [i0s[k]])
                        for k in range(2)
                        for d in range(EMB_DIM)
                    }
                    res = {k: e0s[k] + ws[k[0]] * (e1s[k] - e0s[k]) for k in e0s}
                    for k in range(2):
                        for d in range(EMB_DIM):
                            st[lvl][d][pl.ds(sgs[k], 16)] = res[(k, d)]

            for ch in range(F // 8):
                out_copy(c, ch, ob, osem).start()
        return carry

    lax.fori_loop(0, NCH // 2, super_body, 0)

    for b in range(2):
        for ch in range(F // 8):
            out_copy(NCH - 2 + b, ch, out_bufs[b], out_sems[b]).wait()


_SC_CALL = None


def _get_sc_call():
    global _SC_CALL
    if _SC_CALL is None:
        mesh = plsc.VectorSubcoreMesh(
            core_axis_name="c", subcore_axis_name="s", num_cores=NC, num_subcores=NS
        )
        _SC_CALL = pl.kernel(
            _body,
            out_type=jax.ShapeDtypeStruct((OUT_WORDS,), jnp.float32),
            mesh=mesh,
            compiler_params=pltpu.CompilerParams(
                needs_layout_passes=False, use_tc_tiling_on_sc=False
            ),
            scratch_types=[
                pltpu.VMEM((_GCHUNK * _GBATCH,), jnp.int32),
                pltpu.VMEM((2 * NCW,), jnp.float32),
                pltpu.VMEM((CHUNK,), jnp.float32),
                pltpu.VMEM((CHUNK,), jnp.float32),
                pltpu.VMEM((CHUNK * F,), jnp.float32),
                pltpu.VMEM((CHUNK * F,), jnp.float32),
                pltpu.SemaphoreType.DMA,
                pltpu.SemaphoreType.DMA,
                pltpu.SemaphoreType.DMA,
                pltpu.SemaphoreType.DMA,
            ],
        )
    return _SC_CALL


def kernel(x, tables):
    tab_flat = tables.reshape(NUM_LEVELS, HASHMAP // 128, 128, EMB_DIM).transpose(
        0, 1, 3, 2
    ).reshape(NUM_LEVELS * HASHMAP * EMB_DIM)
    cidx = jnp.asarray(_CIDX)
    out_flat = _get_sc_call()(tab_flat, cidx, x)
    return (
        out_flat.reshape(F // 8, B // 128, 8, 128)
        .transpose(1, 3, 0, 2)
        .reshape(B, F)
    )


# speedup vs baseline: 1.0389x; 1.0389x over previous
import math

import jax
import jax.numpy as jnp
import numpy as np
from jax import lax
from jax.experimental import pallas as pl
from jax.experimental.pallas import tpu as pltpu
from jax.experimental.pallas import tpu_sc as plsc

NUM_LEVELS = 16
MIN_RES = 16
MAX_RES = 2048
EMB_DIM = 4
HASHMAP = 524288
B = 1048576
F = NUM_LEVELS * EMB_DIM

_RES = np.round(
    np.logspace(math.log10(float(MIN_RES)), math.log10(float(MAX_RES)), NUM_LEVELS)
).astype(np.int32)
_RES8 = [((int(r) + 7) // 8) * 8 for r in _RES]
_OFFS = np.concatenate([[0], np.cumsum(_RES8)[:-1]]).astype(np.int32)
R_TOTAL = int(sum(_RES8))
R_PAD = 7424

NC, NS = 2, 16
NW = NC * NS
PT = B // NW
CHUNK = 512
NGRP = CHUNK // 16
PBLK = B // 128


def _tab_word(lvl, r, d):
    return lvl * (HASHMAP * EMB_DIM) + (r // 128) * 512 + d * 128 + (r % 128)


def _compact_word_indices() -> np.ndarray:
    cidx = np.zeros((2 * EMB_DIM * R_PAD,), dtype=np.int64)
    for shift in range(2):
        for lvl in range(NUM_LEVELS):
            r = int(_RES[lvl])
            i = np.arange(r, dtype=np.int64)
            isrc = np.minimum(i + shift, r - 1)
            h = ((isrc * 73856093) ^ (lvl * 19349663)) & (HASHMAP - 1)
            for d in range(EMB_DIM):
                o = shift * EMB_DIM * R_PAD + d * R_PAD + int(_OFFS[lvl])
                cidx[o : o + r] = _tab_word(lvl, h, d)
    return cidx.astype(np.int32)


_CIDX = _compact_word_indices()
NCW = EMB_DIM * R_PAD
_GCHUNK = 128
_GBATCH = 16

_GK = [
    [d * R_PAD + int(_OFFS[lvl]) for d in range(EMB_DIM)] for lvl in range(NUM_LEVELS)
]
_OK = [[((lvl * EMB_DIM + d) // 8) * 4096 + ((lvl * EMB_DIM + d) % 8) * 128
        for d in range(EMB_DIM)] for lvl in range(NUM_LEVELS)]

OUT_WORDS = B * F


NCH = PT // CHUNK
_OBLK = 4096


def _body(
    tab_hbm,
    cidx_hbm,
    x_hbm,
    out_hbm,
    cbuf_v,
    compact_v,
    x_v0,
    x_v1,
    out_v0,
    out_v1,
    sem_g,
    sem_x,
    sem_o0,
    sem_o1,
):
    cid = lax.axis_index("c")
    sid = lax.axis_index("s")
    wid = sid * NC + cid

    def gather_step(j, carry):
        o = j * (_GCHUNK * _GBATCH)
        pltpu.sync_copy(cidx_hbm.at[pl.ds(o, _GCHUNK * _GBATCH)], cbuf_v)
        copies = []
        for b in range(_GBATCH):
            copies.append(
                pltpu.async_copy(
                    tab_hbm.at[cbuf_v.at[pl.ds(b * _GCHUNK, _GCHUNK)]],
                    compact_v.at[pl.ds(o + b * _GCHUNK, _GCHUNK)],
                    sem_g,
                )
            )
        for cp in copies:
            cp.wait()
        return carry

    lax.fori_loop(0, 2 * NCW // (_GCHUNK * _GBATCH), gather_step, 0)

    iota = lax.iota(jnp.int32, 16)
    base_pt = wid * PT
    base_blk = wid * (PT // 128)

    x_bufs = (x_v0, x_v1)
    out_bufs = (out_v0, out_v1)
    out_sems = (sem_o0, sem_o1)

    def x_copy(c, buf):
        return pltpu.make_async_copy(
            x_hbm.at[pl.ds(base_pt + c * CHUNK, CHUNK)], buf, sem_x
        )

    def out_copy(c, ch, buf, sem):
        dst = ch * (PBLK * 1024) + (base_blk + c * (CHUNK // 128)) * 1024
        return pltpu.make_async_copy(
            buf.at[pl.ds(ch * _OBLK, _OBLK)], out_hbm.at[pl.ds(dst, _OBLK)], sem
        )

    x_copy(0, x_v0).start()

    def super_body(cc, carry):
        for b in range(2):
            c = cc * 2 + b
            xb = x_bufs[b]
            ob = out_bufs[b]
            osem = out_sems[b]

            x_copy(c, xb).wait()

            @pl.when(c + 1 < NCH)
            def _prefetch():
                x_copy(c + 1, x_bufs[1 - b]).start()

            @pl.when(cc >= 1)
            def _drain_prev():
                for ch in range(F // 8):
                    out_copy(c, ch, ob, osem).wait()

            g0 = [
                [compact_v.at[pl.ds(_GK[lvl][d], NCW - _GK[lvl][d])] for d in range(EMB_DIM)]
                for lvl in range(NUM_LEVELS)
            ]
            g1 = [
                [
                    compact_v.at[pl.ds(NCW + _GK[lvl][d], NCW - _GK[lvl][d])]
                    for d in range(EMB_DIM)
                ]
                for lvl in range(NUM_LEVELS)
            ]
            st = [
                [ob.at[pl.ds(_OK[lvl][d], CHUNK * F - _OK[lvl][d])] for d in range(EMB_DIM)]
                for lvl in range(NUM_LEVELS)
            ]

            @plsc.parallel_loop(0, NGRP, 1, unroll=2)
            def grp_body(g):
                xv = xb[pl.ds(g * 16, 16)]
                xc = jnp.minimum(jnp.maximum(xv, jnp.float32(0.0)), jnp.float32(1.0))
                sg = (g >> 3) * 1024 + (g & 7) * 16
                for lp in range(NUM_LEVELS // 2):
                    lvls = (2 * lp, 2 * lp + 1)
                    i0s, ws = {}, {}
                    for lvl in lvls:
                        rl = int(_RES[lvl])
                        t = xc * jnp.float32(rl - 1)
                        i0s[lvl] = t.astype(jnp.int32)
                        ws[lvl] = t - i0s[lvl].astype(jnp.float32)
                    e0s = {
                        (lvl, d): plsc.load_gather(g0[lvl][d], [i0s[lvl]])
                        for lvl in lvls
                        for d in range(EMB_DIM)
                    }
                    e1s = {
                        (lvl, d): plsc.load_gather(g1[lvl][d], [i0s[lvl]])
                        for lvl in lvls
                        for d in range(EMB_DIM)
                    }
                    res = {
                        k: e0s[k] + ws[k[0]] * (e1s[k] - e0s[k]) for k in e0s
                    }
                    for lvl in lvls:
                        for d in range(EMB_DIM):
                            st[lvl][d][pl.ds(sg, 16)] = res[(lvl, d)]

            for ch in range(F // 8):
                out_copy(c, ch, ob, osem).start()
        return carry

    lax.fori_loop(0, NCH // 2, super_body, 0)

    for b in range(2):
        for ch in range(F // 8):
            out_copy(NCH - 2 + b, ch, out_bufs[b], out_sems[b]).wait()


_SC_CALL = None


def _get_sc_call():
    global _SC_CALL
    if _SC_CALL is None:
        mesh = plsc.VectorSubcoreMesh(
            core_axis_name="c", subcore_axis_name="s", num_cores=NC, num_subcores=NS
        )
        _SC_CALL = pl.kernel(
            _body,
            out_type=jax.ShapeDtypeStruct((OUT_WORDS,), jnp.float32),
            mesh=mesh,
            compiler_params=pltpu.CompilerParams(
                needs_layout_passes=False, use_tc_tiling_on_sc=False
            ),
            scratch_types=[
                pltpu.VMEM((_GCHUNK * _GBATCH,), jnp.int32),
                pltpu.VMEM((2 * NCW,), jnp.float32),
                pltpu.VMEM((CHUNK,), jnp.float32),
                pltpu.VMEM((CHUNK,), jnp.float32),
                pltpu.VMEM((CHUNK * F,), jnp.float32),
                pltpu.VMEM((CHUNK * F,), jnp.float32),
                pltpu.SemaphoreType.DMA,
                pltpu.SemaphoreType.DMA,
                pltpu.SemaphoreType.DMA,
                pltpu.SemaphoreType.DMA,
            ],
        )
    return _SC_CALL


def kernel(x, tables):
    tab_flat = tables.reshape(NUM_LEVELS, HASHMAP // 128, 128, EMB_DIM).transpose(
        0, 1, 3, 2
    ).reshape(NUM_LEVELS * HASHMAP * EMB_DIM)
    cidx = jnp.asarray(_CIDX)
    out_flat = _get_sc_call()(tab_flat, cidx, x)
    return (
        out_flat.reshape(F // 8, B // 128, 8, 128)
        .transpose(1, 3, 0, 2)
        .reshape(B, F)
    )
